# baseline (speedup 1.0000x reference)
import functools

import jax
import jax.numpy as jnp
from jax import lax
from jax.experimental import pallas as pl
from jax.experimental.pallas import tpu as pltpu
from jax.experimental.pallas import tpu_sc as plsc

_N = 10000
_E = 320000
_D = 128
_NC = 2
_NS = 16
_NW = _NC * _NS
_CH = 128
_NCHUNKS = _E // _CH

_RPT = 632
_LAST = _N - (_NS - 1) * _RPT


def _run(f):
    return f()


def _agg_body(two_streams, table, eidx, zeros, out,
              ev0, ev1, ev2, dv0, dv1, dv2, sv0, sv1, sv2,
              rows0, rows1, rows2,
              isem0, isem1, isem2,
              gsem0, gsem1, gsem2, asem0, asem1, asem2, acc_sh):
    c = lax.axis_index("c")
    s = lax.axis_index("s")

    @pl.when(s < _NS - 1)
    def _():
        b = s * _RPT
        pltpu.sync_copy(zeros.at[pl.ds(b, _RPT)], acc_sh.at[pl.ds(b, _RPT)])

    @pl.when(s == _NS - 1)
    def _():
        b = (_NS - 1) * _RPT
        pltpu.sync_copy(zeros.at[pl.ds(b, _LAST)], acc_sh.at[pl.ds(b, _LAST)])

    plsc.subcore_barrier()

    if two_streams:
        nk = jnp.where(s < _NCHUNKS - _NS * (_NCHUNKS // _NS),
                       _NCHUNKS // _NS + 1, _NCHUNKS // _NS)
        stride = _NS
        first = s
        row_base = 0
    else:
        wid = s * _NC + c
        nk = jnp.where(wid < _NCHUNKS - _NW * (_NCHUNKS // _NW),
                       _NCHUNKS // _NW + 1, _NCHUNKS // _NW)
        stride = _NW
        first = wid
        row_base = 0

    ev = [ev0, ev1, ev2]
    dv = [dv0, dv1, dv2]
    sv = [sv0, sv1, sv2]
    rv = [rows0, rows1, rows2]
    isem = [isem0, isem1, isem2]
    gsem = [gsem0, gsem1, gsem2]
    asem = [asem0, asem1, asem2]

    def fire_idx(m, slot):
        r0 = row_base + 2 * (first + stride * m)
        pltpu.async_copy(eidx.at[pl.ds(r0, 2)], ev[slot], isem[slot])

    def fire_gather(slot):
        if two_streams:
            pltpu.async_copy(table.at[c].at[ev[slot].at[0]], rv[slot],
                             gsem[slot])
        else:
            pltpu.async_copy(table.at[ev[slot].at[0]], rv[slot], gsem[slot])

    def drain_idx(slot):
        pltpu.make_async_copy(eidx.at[pl.ds(0, 2)], ev[slot],
                              isem[slot]).wait()

    def drain(sem, slot):
        pltpu.make_async_copy(zeros.at[pl.ds(0, _CH)], rv[slot],
                              sem[slot]).wait()

    def process(m, slot, warmup):
        if not warmup:
            drain(asem, (slot + 1) % 3)

        idx_pf = _run if warmup else pl.when(m + 2 < nk)

        @idx_pf
        def _():
            fire_idx(m + 2, (slot + 2) % 3)

        g_pf = _run if warmup else pl.when(m + 1 < nk)

        @g_pf
        def _():
            drain_idx((slot + 1) % 3)
            fire_gather((slot + 1) % 3)

        drain(gsem, slot)
        for tt in range(_CH // 16):
            dv[slot][pl.ds(16 * tt, 16)] = ev[slot][1, pl.ds(16 * tt, 16)]
        pltpu.async_copy(rv[slot], acc_sh.at[dv[slot]], asem[slot],
                         add=True)

    fire_idx(0, 0)
    fire_idx(1, 1)
    drain_idx(0)
    fire_gather(0)
    process(0, 0, warmup=True)
    process(1, 1, warmup=True)

    @pl.loop(0, nk // 3)
    def _(kk):
        m0 = 2 + 3 * kk
        for i in range(3):
            @pl.when(m0 + i < nk)
            def _(i=i):
                process(m0 + i, (2 + i) % 3, warmup=False)

    for tail in (2, 1):
        for jj in range(3):
            @pl.when((nk - tail) % 3 == jj)
            def _(jj=jj):
                drain(asem, jj)

    plsc.subcore_barrier()

    @pl.when(s < _NS - 1)
    def _():
        b = s * _RPT
        pltpu.sync_copy(acc_sh.at[pl.ds(b, _RPT)],
                        out.at[pl.ds(c * _N + b, _RPT)])

    @pl.when(s == _NS - 1)
    def _():
        b = (_NS - 1) * _RPT
        pltpu.sync_copy(acc_sh.at[pl.ds(b, _LAST)],
                        out.at[pl.ds(c * _N + b, _LAST)])


def _sc_agg(table, eidx, zeros, two_streams):
    body = functools.partial(_agg_body, two_streams)
    return pl.kernel(
        body,
        out_type=jax.ShapeDtypeStruct((2 * _N, _D), jnp.float32),
        mesh=plsc.VectorSubcoreMesh(core_axis_name="c", subcore_axis_name="s",
                                    num_cores=_NC, num_subcores=_NS),
        scratch_types=(
            [pltpu.VMEM((2, _CH), jnp.int32) for _ in range(3)]
            + [pltpu.VMEM((_CH,), jnp.int32) for _ in range(6)]
            + [pltpu.VMEM((_CH, _D), jnp.float32) for _ in range(3)]
            + [pltpu.SemaphoreType.DMA for _ in range(9)]
            + [pltpu.VMEM_SHARED((_N, _D), jnp.float32)]
        ),
    )(table, eidx, zeros)


def _tc1_body(x_ref, p_ref, wmu_ref, bmu_ref, wlv_ref, blv_ref, out_ref):
    h = x_ref[...] + p_ref[0] + p_ref[1]
    mu = jnp.dot(h, wmu_ref[...], preferred_element_type=jnp.float32)
    lv = jnp.dot(h, wlv_ref[...], preferred_element_type=jnp.float32)
    out_ref[0] = jnp.maximum(mu + bmu_ref[...], 0.0)
    out_ref[1] = jnp.maximum(lv + blv_ref[...], 0.0)


def _tc2_body(h_ref, a_ref, wmu_ref, bmu_ref, wlv_ref, blv_ref,
              mu_ref, lv_ref):
    hmu = h_ref[0] + a_ref[0]
    hlv = h_ref[1] + a_ref[1]
    mu_ref[...] = (jnp.dot(hmu, wmu_ref[...], preferred_element_type=jnp.float32)
                   + bmu_ref[...])
    lv_ref[...] = (jnp.dot(hlv, wlv_ref[...], preferred_element_type=jnp.float32)
                   + blv_ref[...])


_BN = 1000

_W_SPEC = pl.BlockSpec((_D, _D), lambda i: (0, 0))
_B_SPEC = pl.BlockSpec((1, _D), lambda i: (0, 0))


def _tc1(x, parts, wmu, bmu, wlv, blv):
    return pl.pallas_call(
        _tc1_body,
        grid=(_N // _BN,),
        in_specs=[
            pl.BlockSpec((_BN, _D), lambda i: (i, 0)),
            pl.BlockSpec((2, _BN, _D), lambda i: (0, i, 0)),
            _W_SPEC, _B_SPEC, _W_SPEC, _B_SPEC,
        ],
        out_specs=pl.BlockSpec((2, _BN, _D), lambda i: (0, i, 0)),
        out_shape=jax.ShapeDtypeStruct((2, _N, _D), jnp.float32),
    )(x, parts, wmu, bmu, wlv, blv)


def _tc2(h01, aggs, wmu, bmu, wlv, blv):
    return pl.pallas_call(
        _tc2_body,
        grid=(_N // _BN,),
        in_specs=[
            pl.BlockSpec((2, _BN, _D), lambda i: (0, i, 0)),
            pl.BlockSpec((2, _BN, _D), lambda i: (0, i, 0)),
            _W_SPEC, _B_SPEC, _W_SPEC, _B_SPEC,
        ],
        out_specs=[
            pl.BlockSpec((_BN, _D), lambda i: (i, 0)),
            pl.BlockSpec((_BN, _D), lambda i: (i, 0)),
        ],
        out_shape=[
            jax.ShapeDtypeStruct((_N, _D), jnp.float32),
            jax.ShapeDtypeStruct((_N, _D), jnp.float32),
        ],
    )(h01, aggs, wmu, bmu, wlv, blv)


def _pack_eidx(src, dst):
    return jnp.stack([src.reshape(_NCHUNKS, _CH),
                      dst.reshape(_NCHUNKS, _CH)],
                     axis=1).reshape(2 * _NCHUNKS, _CH)


def kernel(x, edge_index, W_mu0, b_mu0, W_lv0, b_lv0,
           W_mu1, b_mu1, W_lv1, b_lv1):
    src = edge_index[0]
    dst = edge_index[1]
    zeros = jnp.zeros((_N, _D), jnp.float32)

    eidx_a = _pack_eidx(src, dst)
    parts = _sc_agg(x, eidx_a, zeros, two_streams=False)
    parts = parts.reshape(2, _N, _D)
    h01 = _tc1(x, parts, W_mu0, b_mu0.reshape(1, _D),
               W_lv0, b_lv0.reshape(1, _D))

    aggs = _sc_agg(h01, eidx_a, zeros, two_streams=True)
    mu, lv = _tc2(h01, aggs.reshape(2, _N, _D), W_mu1, b_mu1.reshape(1, _D),
                  W_lv1, b_lv1.reshape(1, _D))
    return (mu, lv)


# speedup vs baseline: 1.0033x; 1.0033x over previous
import functools

import jax
import jax.numpy as jnp
from jax import lax
from jax.experimental import pallas as pl
from jax.experimental.pallas import tpu as pltpu
from jax.experimental.pallas import tpu_sc as plsc

_N = 10000
_E = 320000
_D = 128
_NC = 2
_NS = 16
_NW = _NC * _NS
_CH = 128
_NCHUNKS = _E // _CH

_RPT = 632
_LAST = _N - (_NS - 1) * _RPT


def _run(f):
    return f()


def _agg_body(two_streams, table, eidx, zeros, out,
              ev0, ev1, ev2, dv0, dv1, dv2, rows0, rows1, rows2,
              isem0, isem1, isem2,
              gsem0, gsem1, gsem2, asem0, asem1, asem2, acc_sh):
    c = lax.axis_index("c")
    s = lax.axis_index("s")

    @pl.when(s < _NS - 1)
    def _():
        b = s * _RPT
        pltpu.sync_copy(zeros.at[pl.ds(b, _RPT)], acc_sh.at[pl.ds(b, _RPT)])

    @pl.when(s == _NS - 1)
    def _():
        b = (_NS - 1) * _RPT
        pltpu.sync_copy(zeros.at[pl.ds(b, _LAST)], acc_sh.at[pl.ds(b, _LAST)])

    plsc.subcore_barrier()

    if two_streams:
        nk = jnp.where(s < _NCHUNKS - _NS * (_NCHUNKS // _NS),
                       _NCHUNKS // _NS + 1, _NCHUNKS // _NS)
        stride = _NS
        first = s
        row_base = 0
    else:
        wid = s * _NC + c
        nk = jnp.where(wid < _NCHUNKS - _NW * (_NCHUNKS // _NW),
                       _NCHUNKS // _NW + 1, _NCHUNKS // _NW)
        stride = _NW
        first = wid
        row_base = 0

    ev = [ev0, ev1, ev2]
    dv = [dv0, dv1, dv2]
    rv = [rows0, rows1, rows2]
    isem = [isem0, isem1, isem2]
    gsem = [gsem0, gsem1, gsem2]
    asem = [asem0, asem1, asem2]

    def fire_idx(m, slot):
        r0 = row_base + 2 * (first + stride * m)
        pltpu.async_copy(eidx.at[pl.ds(r0, 2)], ev[slot], isem[slot])

    def fire_gather(slot):
        if two_streams:
            pltpu.async_copy(table.at[c].at[ev[slot].at[0]], rv[slot],
                             gsem[slot])
        else:
            pltpu.async_copy(table.at[ev[slot].at[0]], rv[slot], gsem[slot])

    def drain_idx(slot):
        pltpu.make_async_copy(eidx.at[pl.ds(0, 2)], ev[slot],
                              isem[slot]).wait()

    def drain(sem, slot):
        pltpu.make_async_copy(zeros.at[pl.ds(0, _CH)], rv[slot],
                              sem[slot]).wait()

    def process(m, slot, warmup):
        if not warmup:
            drain(asem, (slot + 1) % 3)

        idx_pf = _run if warmup else pl.when(m + 2 < nk)

        @idx_pf
        def _():
            fire_idx(m + 2, (slot + 2) % 3)

        g_pf = _run if warmup else pl.when(m + 1 < nk)

        @g_pf
        def _():
            drain_idx((slot + 1) % 3)
            fire_gather((slot + 1) % 3)

        drain(gsem, slot)
        for tt in range(_CH // 16):
            dv[slot][pl.ds(16 * tt, 16)] = ev[slot][1, pl.ds(16 * tt, 16)]
        pltpu.async_copy(rv[slot], acc_sh.at[dv[slot]], asem[slot],
                         add=True)

    fire_idx(0, 0)
    fire_idx(1, 1)
    drain_idx(0)
    fire_gather(0)
    process(0, 0, warmup=True)
    process(1, 1, warmup=True)

    @pl.loop(0, nk // 3)
    def _(kk):
        m0 = 2 + 3 * kk
        for i in range(3):
            @pl.when(m0 + i < nk)
            def _(i=i):
                process(m0 + i, (2 + i) % 3, warmup=False)

    for tail in (2, 1):
        for jj in range(3):
            @pl.when((nk - tail) % 3 == jj)
            def _(jj=jj):
                drain(asem, jj)

    plsc.subcore_barrier()

    @pl.when(s < _NS - 1)
    def _():
        b = s * _RPT
        pltpu.sync_copy(acc_sh.at[pl.ds(b, _RPT)],
                        out.at[pl.ds(c * _N + b, _RPT)])

    @pl.when(s == _NS - 1)
    def _():
        b = (_NS - 1) * _RPT
        pltpu.sync_copy(acc_sh.at[pl.ds(b, _LAST)],
                        out.at[pl.ds(c * _N + b, _LAST)])


def _sc_agg(table, eidx, zeros, two_streams):
    body = functools.partial(_agg_body, two_streams)
    return pl.kernel(
        body,
        out_type=jax.ShapeDtypeStruct((2 * _N, _D), jnp.float32),
        mesh=plsc.VectorSubcoreMesh(core_axis_name="c", subcore_axis_name="s",
                                    num_cores=_NC, num_subcores=_NS),
        scratch_types=(
            [pltpu.VMEM((2, _CH), jnp.int32) for _ in range(3)]
            + [pltpu.VMEM((_CH,), jnp.int32) for _ in range(3)]
            + [pltpu.VMEM((_CH, _D), jnp.float32) for _ in range(3)]
            + [pltpu.SemaphoreType.DMA for _ in range(9)]
            + [pltpu.VMEM_SHARED((_N, _D), jnp.float32)]
        ),
    )(table, eidx, zeros)


def _tc1_body(x_ref, p_ref, wmu_ref, bmu_ref, wlv_ref, blv_ref, out_ref):
    h = x_ref[...] + p_ref[0] + p_ref[1]
    mu = jnp.dot(h, wmu_ref[...], preferred_element_type=jnp.float32)
    lv = jnp.dot(h, wlv_ref[...], preferred_element_type=jnp.float32)
    out_ref[0] = jnp.maximum(mu + bmu_ref[...], 0.0)
    out_ref[1] = jnp.maximum(lv + blv_ref[...], 0.0)


def _tc2_body(h_ref, a_ref, wmu_ref, bmu_ref, wlv_ref, blv_ref,
              mu_ref, lv_ref):
    hmu = h_ref[0] + a_ref[0]
    hlv = h_ref[1] + a_ref[1]
    mu_ref[...] = (jnp.dot(hmu, wmu_ref[...], preferred_element_type=jnp.float32)
                   + bmu_ref[...])
    lv_ref[...] = (jnp.dot(hlv, wlv_ref[...], preferred_element_type=jnp.float32)
                   + blv_ref[...])


_BN = 1000

_W_SPEC = pl.BlockSpec((_D, _D), lambda i: (0, 0))
_B_SPEC = pl.BlockSpec((1, _D), lambda i: (0, 0))


def _tc1(x, parts, wmu, bmu, wlv, blv):
    return pl.pallas_call(
        _tc1_body,
        grid=(_N // _BN,),
        in_specs=[
            pl.BlockSpec((_BN, _D), lambda i: (i, 0)),
            pl.BlockSpec((2, _BN, _D), lambda i: (0, i, 0)),
            _W_SPEC, _B_SPEC, _W_SPEC, _B_SPEC,
        ],
        out_specs=pl.BlockSpec((2, _BN, _D), lambda i: (0, i, 0)),
        out_shape=jax.ShapeDtypeStruct((2, _N, _D), jnp.float32),
    )(x, parts, wmu, bmu, wlv, blv)


def _tc2(h01, aggs, wmu, bmu, wlv, blv):
    return pl.pallas_call(
        _tc2_body,
        grid=(_N // _BN,),
        in_specs=[
            pl.BlockSpec((2, _BN, _D), lambda i: (0, i, 0)),
            pl.BlockSpec((2, _BN, _D), lambda i: (0, i, 0)),
            _W_SPEC, _B_SPEC, _W_SPEC, _B_SPEC,
        ],
        out_specs=[
            pl.BlockSpec((_BN, _D), lambda i: (i, 0)),
            pl.BlockSpec((_BN, _D), lambda i: (i, 0)),
        ],
        out_shape=[
            jax.ShapeDtypeStruct((_N, _D), jnp.float32),
            jax.ShapeDtypeStruct((_N, _D), jnp.float32),
        ],
    )(h01, aggs, wmu, bmu, wlv, blv)


def _pack_eidx(src, dst):
    return jnp.stack([src.reshape(_NCHUNKS, _CH),
                      dst.reshape(_NCHUNKS, _CH)],
                     axis=1).reshape(2 * _NCHUNKS, _CH)


def kernel(x, edge_index, W_mu0, b_mu0, W_lv0, b_lv0,
           W_mu1, b_mu1, W_lv1, b_lv1):
    src = edge_index[0]
    dst = edge_index[1]
    zeros = jnp.zeros((_N, _D), jnp.float32)

    eidx_a = _pack_eidx(src, dst)
    parts = _sc_agg(x, eidx_a, zeros, two_streams=False)
    parts = parts.reshape(2, _N, _D)
    h01 = _tc1(x, parts, W_mu0, b_mu0.reshape(1, _D),
               W_lv0, b_lv0.reshape(1, _D))

    aggs = _sc_agg(h01, eidx_a, zeros, two_streams=True)
    mu, lv = _tc2(h01, aggs.reshape(2, _N, _D), W_mu1, b_mu1.reshape(1, _D),
                  W_lv1, b_lv1.reshape(1, _D))
    return (mu, lv)
